# baseline (speedup 1.0000x reference)
import functools

import jax
import jax.numpy as jnp
from jax import lax
from jax.experimental import pallas as pl
from jax.experimental.pallas import tpu as pltpu
from jax.experimental.pallas import tpu_sc as plsc

_N = 10000
_E = 320000
_H = 128
_PDGS = (1, 2, 11, 13, 22, 130, 211)
_NTILES = 32
_FPT = _H // _NTILES
_CH = 6400
_G = 8
_LANES = 16


def _elu(x):
    return jnp.where(x > 0, x, jnp.exp(jnp.minimum(x, 0.0)) - 1.0)


def _bn(x, g, b, eps=1e-5):
    m = jnp.mean(x, axis=0)
    v = jnp.mean((x - m) ** 2, axis=0)
    return g * (x - m) * lax.rsqrt(v + eps) + b


def _enc_body(x_cont_ref, x_cat_ref, emb_charge_ref, emb_pdgid_ref,
              w_cont_ref, b_cont_ref, w_cat_ref, b_cat_ref,
              w_enc_ref, b_enc_ref, g_all_ref, b_all_ref,
              w_msg_ref, b_msg_ref,
              emb_ref, a_ref, bt_ref):
    xc = x_cont_ref[...]
    emb_cont = _elu(jnp.dot(xc, w_cont_ref[...],
                            preferred_element_type=jnp.float32) + b_cont_ref[...])

    cat = x_cat_ref[...]
    chrg = cat[:, 1:2] + 1
    pdg = jnp.abs(cat[:, 0:1])
    for i, p in enumerate(_PDGS):
        pdg = jnp.where(pdg == p, jnp.full_like(pdg, i), pdg)

    emb_chrg = jnp.zeros((_N, _H // 4), jnp.float32)
    for k in range(3):
        emb_chrg += jnp.where(chrg == k, 1.0, 0.0) * emb_charge_ref[k, :][None, :]
    emb_pdg = jnp.zeros((_N, _H // 4), jnp.float32)
    for k in range(7):
        emb_pdg += jnp.where(pdg == k, 1.0, 0.0) * emb_pdgid_ref[k, :][None, :]

    w_cat = w_cat_ref[...]
    emb_cat = _elu(jnp.dot(emb_chrg, w_cat[:_H // 4, :],
                           preferred_element_type=jnp.float32)
                   + jnp.dot(emb_pdg, w_cat[_H // 4:, :],
                             preferred_element_type=jnp.float32)
                   + b_cat_ref[...])

    w_enc = w_enc_ref[...]
    enc = _elu(jnp.dot(emb_cat, w_enc[:_H // 2, :],
                       preferred_element_type=jnp.float32)
               + jnp.dot(emb_cont, w_enc[_H // 2:, :],
                         preferred_element_type=jnp.float32)
               + b_enc_ref[...])
    emb = _bn(enc, g_all_ref[...], b_all_ref[...])
    emb_ref[...] = emb

    w_msg = w_msg_ref[...]
    wt = w_msg[:_H, :]
    wb = w_msg[_H:, :]
    a_ref[...] = jnp.dot(emb, wt - wb, preferred_element_type=jnp.float32) + b_msg_ref[...]
    bt_ref[...] = jnp.dot(emb, wb, preferred_element_type=jnp.float32).T


_enc_call = pl.pallas_call(
    _enc_body,
    out_shape=[
        jax.ShapeDtypeStruct((_N, _H), jnp.float32),
        jax.ShapeDtypeStruct((_N, _H), jnp.float32),
        jax.ShapeDtypeStruct((_H, _N), jnp.float32),
    ],
)


def _segmax_body(bt_hbm, src_hbm, dst_hbm, out_hbm, b_buf, m_buf, s_buf, d_buf):
    cid = lax.axis_index("c")
    sid = lax.axis_index("s")
    wid = sid * 2 + cid
    f0 = wid * _FPT

    pltpu.sync_copy(bt_hbm.at[pl.ds(f0, _FPT), :], b_buf)

    neg = jnp.full((_LANES,), -jnp.inf, jnp.float32)
    def _init(i, c):
        for f in range(_FPT):
            m_buf[f, pl.ds(i * _LANES, _LANES)] = neg
        return c
    lax.fori_loop(0, _N // _LANES, _init, 0)

    fvecs = [jnp.full((_LANES,), f, jnp.int32) for f in range(_FPT)]

    def _chunk(ci, c):
        base = ci * _CH
        pltpu.sync_copy(src_hbm.at[pl.ds(base, _CH)], s_buf)
        pltpu.sync_copy(dst_hbm.at[pl.ds(base, _CH)], d_buf)

        def _vec(vi, cc):
            base_v = vi * (_G * _LANES)
            ds = [d_buf[pl.ds(base_v + g * _LANES, _LANES)] for g in range(_G)]
            ss = [s_buf[pl.ds(base_v + g * _LANES, _LANES)] for g in range(_G)]
            cms = [plsc.scan_count(d) for d in ds]
            bss = [[plsc.load_gather(b_buf, [fv, s]) for fv in fvecs]
                   for s in ss]
            clean = cms[0][1]
            for _, l in cms[1:]:
                clean = clean & l
            for (cnt, _), d, bs in zip(cms, ds, bss):
                msk1 = cnt == 1
                for fv, b in zip(fvecs, bs):
                    cur = plsc.load_gather(m_buf, [fv, d])
                    plsc.store_scatter(m_buf, [fv, d], jnp.maximum(cur, b),
                                       mask=msk1)

            def _slow(_):
                for (cnt, _), d, bs in zip(cms, ds, bss):
                    nmax = jnp.max(cnt)

                    def _round(r, c2):
                        msk = cnt == r
                        for fv, b in zip(fvecs, bs):
                            cur = plsc.load_gather(m_buf, [fv, d])
                            plsc.store_scatter(m_buf, [fv, d],
                                               jnp.maximum(cur, b), mask=msk)
                        return c2
                    lax.fori_loop(2, nmax + 1, _round, 0)
                return 0
            lax.cond(jnp.all(clean), lambda _: 0, _slow, 0)
            return cc
        lax.fori_loop(0, _CH // (_G * _LANES), _vec, 0)
        return c

    lax.fori_loop(0, _E // _CH, _chunk, 0)

    pltpu.sync_copy(m_buf, out_hbm.at[pl.ds(f0, _FPT), :])


_segmax_call = pl.kernel(
    _segmax_body,
    out_type=jax.ShapeDtypeStruct((_H, _N), jnp.float32),
    mesh=plsc.VectorSubcoreMesh(core_axis_name="c", subcore_axis_name="s"),
    compiler_params=pltpu.CompilerParams(needs_layout_passes=False),
    scratch_types=[
        pltpu.VMEM((_FPT, _N), jnp.float32),
        pltpu.VMEM((_FPT, _N), jnp.float32),
        pltpu.VMEM((_CH,), jnp.int32),
        pltpu.VMEM((_CH,), jnp.int32),
    ],
)


def _tail_body(emb_ref, a_ref, mt_ref, g1_ref, b1_ref,
               w_o1_ref, b_o1_ref, w_o2_ref, b_o2_ref, out_ref):
    m = mt_ref[...].T
    agg = jnp.where(m > -jnp.inf, a_ref[...] + m, 0.0)
    emb2 = emb_ref[...] + _bn(agg, g1_ref[...], b1_ref[...])
    h = _elu(jnp.dot(emb2, w_o1_ref[...], preferred_element_type=jnp.float32)
             + b_o1_ref[...])
    out_ref[...] = jnp.dot(h, w_o2_ref[...],
                           preferred_element_type=jnp.float32) + b_o2_ref[...]


_tail_call = pl.pallas_call(
    _tail_body,
    out_shape=jax.ShapeDtypeStruct((_N, 1), jnp.float32),
)


def kernel(x_cont, x_cat, edge_index, batch, emb_charge, emb_pdgid,
           W_cont, b_cont, W_cat, b_cat, W_enc, b_enc, g_all, b_all,
           W_msg, b_msg, g_bn1, b_bn1, W_o1, b_o1, W_o2, b_o2):
    del batch
    emb, a, bt = _enc_call(x_cont, x_cat, emb_charge, emb_pdgid,
                           W_cont, b_cont, W_cat, b_cat, W_enc, b_enc,
                           g_all, b_all, W_msg, b_msg)
    mt = _segmax_call(bt, edge_index[0], edge_index[1])
    out = _tail_call(emb, a, mt, g_bn1, b_bn1, W_o1, b_o1, W_o2, b_o2)
    return out.squeeze(-1)


# speedup vs baseline: 1.2055x; 1.2055x over previous
import functools

import jax
import jax.numpy as jnp
from jax import lax
from jax.experimental import pallas as pl
from jax.experimental.pallas import tpu as pltpu
from jax.experimental.pallas import tpu_sc as plsc

_N = 10000
_E = 320000
_H = 128
_PDGS = (1, 2, 11, 13, 22, 130, 211)
_NTILES = 32
_FPT = _H // _NTILES
_CH = 6400
_G = 4
_LANES = 16


def _elu(x):
    return jnp.where(x > 0, x, jnp.exp(jnp.minimum(x, 0.0)) - 1.0)


def _bn(x, g, b, eps=1e-5):
    m = jnp.mean(x, axis=0)
    v = jnp.mean((x - m) ** 2, axis=0)
    return g * (x - m) * lax.rsqrt(v + eps) + b


def _enc_body(x_cont_ref, x_cat_ref, emb_charge_ref, emb_pdgid_ref,
              w_cont_ref, b_cont_ref, w_cat_ref, b_cat_ref,
              w_enc_ref, b_enc_ref, g_all_ref, b_all_ref,
              w_msg_ref, b_msg_ref,
              emb_ref, a_ref, bt_ref):
    xc = x_cont_ref[...]
    emb_cont = _elu(jnp.dot(xc, w_cont_ref[...],
                            preferred_element_type=jnp.float32) + b_cont_ref[...])

    cat = x_cat_ref[...]
    chrg = cat[:, 1:2] + 1
    pdg = jnp.abs(cat[:, 0:1])
    for i, p in enumerate(_PDGS):
        pdg = jnp.where(pdg == p, jnp.full_like(pdg, i), pdg)

    emb_chrg = jnp.zeros((_N, _H // 4), jnp.float32)
    for k in range(3):
        emb_chrg += jnp.where(chrg == k, 1.0, 0.0) * emb_charge_ref[k, :][None, :]
    emb_pdg = jnp.zeros((_N, _H // 4), jnp.float32)
    for k in range(7):
        emb_pdg += jnp.where(pdg == k, 1.0, 0.0) * emb_pdgid_ref[k, :][None, :]

    w_cat = w_cat_ref[...]
    emb_cat = _elu(jnp.dot(emb_chrg, w_cat[:_H // 4, :],
                           preferred_element_type=jnp.float32)
                   + jnp.dot(emb_pdg, w_cat[_H // 4:, :],
                             preferred_element_type=jnp.float32)
                   + b_cat_ref[...])

    w_enc = w_enc_ref[...]
    enc = _elu(jnp.dot(emb_cat, w_enc[:_H // 2, :],
                       preferred_element_type=jnp.float32)
               + jnp.dot(emb_cont, w_enc[_H // 2:, :],
                         preferred_element_type=jnp.float32)
               + b_enc_ref[...])
    emb = _bn(enc, g_all_ref[...], b_all_ref[...])
    emb_ref[...] = emb

    w_msg = w_msg_ref[...]
    wt = w_msg[:_H, :]
    wb = w_msg[_H:, :]
    a_ref[...] = jnp.dot(emb, wt - wb, preferred_element_type=jnp.float32) + b_msg_ref[...]
    bt_ref[...] = jnp.dot(emb, wb, preferred_element_type=jnp.float32).T


_enc_call = pl.pallas_call(
    _enc_body,
    out_shape=[
        jax.ShapeDtypeStruct((_N, _H), jnp.float32),
        jax.ShapeDtypeStruct((_N, _H), jnp.float32),
        jax.ShapeDtypeStruct((_H, _N), jnp.float32),
    ],
)


def _segmax_body(bt_hbm, e_hbm, out_hbm, b_buf, m_buf, e0, e1, sem0, sem1):
    cid = lax.axis_index("c")
    sid = lax.axis_index("s")
    wid = sid * 2 + cid
    f0 = wid * _FPT

    pltpu.sync_copy(bt_hbm.at[pl.ds(f0, _FPT), :], b_buf)

    neg = jnp.full((_LANES,), -jnp.inf, jnp.float32)
    def _init(i, c):
        for f in range(_FPT):
            m_buf[f, pl.ds(i * _LANES, _LANES)] = neg
        return c
    lax.fori_loop(0, _N // _LANES, _init, 0)

    fvecs = [jnp.full((_LANES,), f, jnp.int32) for f in range(_FPT)]

    def _start(ci, buf, sem):
        pltpu.async_copy(e_hbm.at[:, pl.ds(ci * _CH, _CH)], buf, sem)

    def _wait(ci, buf, sem):
        pltpu.make_async_copy(e_hbm.at[:, pl.ds(ci * _CH, _CH)], buf,
                              sem).wait()

    def _process(e_buf):
        def _vec(vi, cc):
            base_v = vi * (_G * _LANES)
            ds = [e_buf[1, pl.ds(base_v + g * _LANES, _LANES)]
                  for g in range(_G)]
            ss = [e_buf[0, pl.ds(base_v + g * _LANES, _LANES)]
                  for g in range(_G)]
            cms = [plsc.scan_count(d) for d in ds]
            bss = [[plsc.load_gather(b_buf, [fv, s]) for fv in fvecs]
                   for s in ss]
            clean = cms[0][1]
            for _, l in cms[1:]:
                clean = clean & l
            for (cnt, _), d, bs in zip(cms, ds, bss):
                msk1 = cnt == 1
                for fv, b in zip(fvecs, bs):
                    cur = plsc.load_gather(m_buf, [fv, d])
                    plsc.store_scatter(m_buf, [fv, d], jnp.maximum(cur, b),
                                       mask=msk1)

            def _slow(_):
                for (cnt, _), d, bs in zip(cms, ds, bss):
                    nmax = jnp.max(cnt)

                    def _round(r, c2):
                        msk = cnt == r
                        for fv, b in zip(fvecs, bs):
                            cur = plsc.load_gather(m_buf, [fv, d])
                            plsc.store_scatter(m_buf, [fv, d],
                                               jnp.maximum(cur, b), mask=msk)
                        return c2
                    lax.fori_loop(2, nmax + 1, _round, 0)
                return 0
            lax.cond(jnp.all(clean), lambda _: 0, _slow, 0)
            return cc
        lax.fori_loop(0, _CH // (_G * _LANES), _vec, 0)

    npairs = _E // _CH // 2
    _start(0, e0, sem0)
    _start(1, e1, sem1)

    def _pair(p, c):
        c0 = 2 * p
        _wait(c0, e0, sem0)
        _process(e0)

        @pl.when(p < npairs - 1)
        def _():
            _start(c0 + 2, e0, sem0)
        _wait(c0 + 1, e1, sem1)
        _process(e1)

        @pl.when(p < npairs - 1)
        def _():
            _start(c0 + 3, e1, sem1)
        return c

    lax.fori_loop(0, npairs, _pair, 0)

    pltpu.sync_copy(m_buf, out_hbm.at[pl.ds(f0, _FPT), :])


_segmax_call = pl.kernel(
    _segmax_body,
    out_type=jax.ShapeDtypeStruct((_H, _N), jnp.float32),
    mesh=plsc.VectorSubcoreMesh(core_axis_name="c", subcore_axis_name="s"),
    compiler_params=pltpu.CompilerParams(needs_layout_passes=False),
    scratch_types=[
        pltpu.VMEM((_FPT, _N), jnp.float32),
        pltpu.VMEM((_FPT, _N), jnp.float32),
        pltpu.VMEM((2, _CH), jnp.int32),
        pltpu.VMEM((2, _CH), jnp.int32),
        pltpu.SemaphoreType.DMA,
        pltpu.SemaphoreType.DMA,
    ],
)


def _tail_body(emb_ref, a_ref, mt_ref, g1_ref, b1_ref,
               w_o1_ref, b_o1_ref, w_o2_ref, b_o2_ref, out_ref):
    m = mt_ref[...].T
    agg = jnp.where(m > -jnp.inf, a_ref[...] + m, 0.0)
    emb2 = emb_ref[...] + _bn(agg, g1_ref[...], b1_ref[...])
    h = _elu(jnp.dot(emb2, w_o1_ref[...], preferred_element_type=jnp.float32)
             + b_o1_ref[...])
    out_ref[...] = jnp.dot(h, w_o2_ref[...],
                           preferred_element_type=jnp.float32) + b_o2_ref[...]


_tail_call = pl.pallas_call(
    _tail_body,
    out_shape=jax.ShapeDtypeStruct((_N, 1), jnp.float32),
)


def kernel(x_cont, x_cat, edge_index, batch, emb_charge, emb_pdgid,
           W_cont, b_cont, W_cat, b_cat, W_enc, b_enc, g_all, b_all,
           W_msg, b_msg, g_bn1, b_bn1, W_o1, b_o1, W_o2, b_o2):
    del batch
    emb, a, bt = _enc_call(x_cont, x_cat, emb_charge, emb_pdgid,
                           W_cont, b_cont, W_cat, b_cat, W_enc, b_enc,
                           g_all, b_all, W_msg, b_msg)
    mt = _segmax_call(bt, edge_index)
    out = _tail_call(emb, a, mt, g_bn1, b_bn1, W_o1, b_o1, W_o2, b_o2)
    return out.squeeze(-1)


# speedup vs baseline: 1.2105x; 1.0042x over previous
import functools

import jax
import jax.numpy as jnp
from jax import lax
from jax.experimental import pallas as pl
from jax.experimental.pallas import tpu as pltpu
from jax.experimental.pallas import tpu_sc as plsc

_N = 10000
_E = 320000
_H = 128
_PDGS = (1, 2, 11, 13, 22, 130, 211)
_NTILES = 32
_FPT = _H // _NTILES
_CH = 6400
_G = 5
_LANES = 16


def _elu(x):
    return jnp.where(x > 0, x, jnp.exp(jnp.minimum(x, 0.0)) - 1.0)


def _bn(x, g, b, eps=1e-5):
    m = jnp.mean(x, axis=0)
    v = jnp.mean((x - m) ** 2, axis=0)
    return g * (x - m) * lax.rsqrt(v + eps) + b


def _enc_body(x_cont_ref, x_cat_ref, emb_charge_ref, emb_pdgid_ref,
              w_cont_ref, b_cont_ref, w_cat_ref, b_cat_ref,
              w_enc_ref, b_enc_ref, g_all_ref, b_all_ref,
              w_msg_ref, b_msg_ref,
              emb_ref, a_ref, bt_ref):
    xc = x_cont_ref[...]
    emb_cont = _elu(jnp.dot(xc, w_cont_ref[...],
                            preferred_element_type=jnp.float32) + b_cont_ref[...])

    cat = x_cat_ref[...]
    chrg = cat[:, 1:2] + 1
    pdg = jnp.abs(cat[:, 0:1])
    for i, p in enumerate(_PDGS):
        pdg = jnp.where(pdg == p, jnp.full_like(pdg, i), pdg)

    emb_chrg = jnp.zeros((_N, _H // 4), jnp.float32)
    for k in range(3):
        emb_chrg += jnp.where(chrg == k, 1.0, 0.0) * emb_charge_ref[k, :][None, :]
    emb_pdg = jnp.zeros((_N, _H // 4), jnp.float32)
    for k in range(7):
        emb_pdg += jnp.where(pdg == k, 1.0, 0.0) * emb_pdgid_ref[k, :][None, :]

    w_cat = w_cat_ref[...]
    emb_cat = _elu(jnp.dot(emb_chrg, w_cat[:_H // 4, :],
                           preferred_element_type=jnp.float32)
                   + jnp.dot(emb_pdg, w_cat[_H // 4:, :],
                             preferred_element_type=jnp.float32)
                   + b_cat_ref[...])

    w_enc = w_enc_ref[...]
    enc = _elu(jnp.dot(emb_cat, w_enc[:_H // 2, :],
                       preferred_element_type=jnp.float32)
               + jnp.dot(emb_cont, w_enc[_H // 2:, :],
                         preferred_element_type=jnp.float32)
               + b_enc_ref[...])
    emb = _bn(enc, g_all_ref[...], b_all_ref[...])
    emb_ref[...] = emb

    w_msg = w_msg_ref[...]
    wt = w_msg[:_H, :]
    wb = w_msg[_H:, :]
    a_ref[...] = jnp.dot(emb, wt - wb, preferred_element_type=jnp.float32) + b_msg_ref[...]
    bt_ref[...] = jnp.dot(emb, wb, preferred_element_type=jnp.float32).T


_enc_call = pl.pallas_call(
    _enc_body,
    out_shape=[
        jax.ShapeDtypeStruct((_N, _H), jnp.float32),
        jax.ShapeDtypeStruct((_N, _H), jnp.float32),
        jax.ShapeDtypeStruct((_H, _N), jnp.float32),
    ],
)


def _segmax_body(bt_hbm, e_hbm, out_hbm, b_buf, m_buf, e0, e1, sem0, sem1):
    cid = lax.axis_index("c")
    sid = lax.axis_index("s")
    wid = sid * 2 + cid
    f0 = wid * _FPT

    pltpu.sync_copy(bt_hbm.at[pl.ds(f0, _FPT), :], b_buf)

    neg = jnp.full((_LANES,), -jnp.inf, jnp.float32)
    def _init(i, c):
        for f in range(_FPT):
            m_buf[f, pl.ds(i * _LANES, _LANES)] = neg
        return c
    lax.fori_loop(0, _N // _LANES, _init, 0)

    fvecs = [jnp.full((_LANES,), f, jnp.int32) for f in range(_FPT)]

    def _start(ci, buf, sem):
        pltpu.async_copy(e_hbm.at[:, pl.ds(ci * _CH, _CH)], buf, sem)

    def _wait(ci, buf, sem):
        pltpu.make_async_copy(e_hbm.at[:, pl.ds(ci * _CH, _CH)], buf,
                              sem).wait()

    def _process(e_buf):
        def _vec(vi, cc):
            base_v = vi * (_G * _LANES)
            ds = [e_buf[1, pl.ds(base_v + g * _LANES, _LANES)]
                  for g in range(_G)]
            ss = [e_buf[0, pl.ds(base_v + g * _LANES, _LANES)]
                  for g in range(_G)]
            cms = [plsc.scan_count(d) for d in ds]
            bss = [[plsc.load_gather(b_buf, [fv, s]) for fv in fvecs]
                   for s in ss]
            clean = cms[0][1]
            for _, l in cms[1:]:
                clean = clean & l
            for (cnt, _), d, bs in zip(cms, ds, bss):
                msk1 = cnt == 1
                for fv, b in zip(fvecs, bs):
                    cur = plsc.load_gather(m_buf, [fv, d])
                    plsc.store_scatter(m_buf, [fv, d], jnp.maximum(cur, b),
                                       mask=msk1)

            def _slow(_):
                for (cnt, _), d, bs in zip(cms, ds, bss):
                    nmax = jnp.max(cnt)

                    def _round(r, c2):
                        msk = cnt == r
                        for fv, b in zip(fvecs, bs):
                            cur = plsc.load_gather(m_buf, [fv, d])
                            plsc.store_scatter(m_buf, [fv, d],
                                               jnp.maximum(cur, b), mask=msk)
                        return c2
                    lax.fori_loop(2, nmax + 1, _round, 0)
                return 0
            lax.cond(jnp.all(clean), lambda _: 0, _slow, 0)
            return cc
        lax.fori_loop(0, _CH // (_G * _LANES), _vec, 0)

    npairs = _E // _CH // 2
    _start(0, e0, sem0)
    _start(1, e1, sem1)

    def _pair(p, c):
        c0 = 2 * p
        _wait(c0, e0, sem0)
        _process(e0)

        @pl.when(p < npairs - 1)
        def _():
            _start(c0 + 2, e0, sem0)
        _wait(c0 + 1, e1, sem1)
        _process(e1)

        @pl.when(p < npairs - 1)
        def _():
            _start(c0 + 3, e1, sem1)
        return c

    lax.fori_loop(0, npairs, _pair, 0)

    pltpu.sync_copy(m_buf, out_hbm.at[pl.ds(f0, _FPT), :])


_segmax_call = pl.kernel(
    _segmax_body,
    out_type=jax.ShapeDtypeStruct((_H, _N), jnp.float32),
    mesh=plsc.VectorSubcoreMesh(core_axis_name="c", subcore_axis_name="s"),
    compiler_params=pltpu.CompilerParams(needs_layout_passes=False),
    scratch_types=[
        pltpu.VMEM((_FPT, _N), jnp.float32),
        pltpu.VMEM((_FPT, _N), jnp.float32),
        pltpu.VMEM((2, _CH), jnp.int32),
        pltpu.VMEM((2, _CH), jnp.int32),
        pltpu.SemaphoreType.DMA,
        pltpu.SemaphoreType.DMA,
    ],
)


def _tail_body(emb_ref, a_ref, mt_ref, g1_ref, b1_ref,
               w_o1_ref, b_o1_ref, w_o2_ref, b_o2_ref, out_ref):
    m = mt_ref[...].T
    agg = jnp.where(m > -jnp.inf, a_ref[...] + m, 0.0)
    emb2 = emb_ref[...] + _bn(agg, g1_ref[...], b1_ref[...])
    h = _elu(jnp.dot(emb2, w_o1_ref[...], preferred_element_type=jnp.float32)
             + b_o1_ref[...])
    out_ref[...] = jnp.dot(h, w_o2_ref[...],
                           preferred_element_type=jnp.float32) + b_o2_ref[...]


_tail_call = pl.pallas_call(
    _tail_body,
    out_shape=jax.ShapeDtypeStruct((_N, 1), jnp.float32),
)


def kernel(x_cont, x_cat, edge_index, batch, emb_charge, emb_pdgid,
           W_cont, b_cont, W_cat, b_cat, W_enc, b_enc, g_all, b_all,
           W_msg, b_msg, g_bn1, b_bn1, W_o1, b_o1, W_o2, b_o2):
    del batch
    emb, a, bt = _enc_call(x_cont, x_cat, emb_charge, emb_pdgid,
                           W_cont, b_cont, W_cat, b_cat, W_enc, b_enc,
                           g_all, b_all, W_msg, b_msg)
    mt = _segmax_call(bt, edge_index)
    out = _tail_call(emb, a, mt, g_bn1, b_bn1, W_o1, b_o1, W_o2, b_o2)
    return out.squeeze(-1)


# speedup vs baseline: 1.2114x; 1.0008x over previous
import jax
import jax.numpy as jnp
from jax import lax
from jax.experimental import pallas as pl
from jax.experimental.pallas import tpu as pltpu
from jax.experimental.pallas import tpu_sc as plsc

_N = 10000
_E = 320000
_H = 128
_PDGS = (1, 2, 11, 13, 22, 130, 211)
_NTILES = 32
_FPT = _H // _NTILES
_CH = 6400
_G = 5
_LANES = 16


def _elu(x):
    return jnp.where(x > 0, x, jnp.exp(jnp.minimum(x, 0.0)) - 1.0)


def _bn(x, g, b, eps=1e-5):
    m = jnp.mean(x, axis=0)
    v = jnp.mean((x - m) ** 2, axis=0)
    return g * (x - m) * lax.rsqrt(v + eps) + b


def _enc_body(x_cont_ref, x_cat_ref, emb_charge_ref, emb_pdgid_ref,
              w_cont_ref, b_cont_ref, w_cat_ref, b_cat_ref,
              w_enc_ref, b_enc_ref, g_all_ref, b_all_ref,
              w_msg_ref, b_msg_ref,
              emb_ref, a_ref, bt_ref):
    xc = x_cont_ref[...]
    emb_cont = _elu(jnp.dot(xc, w_cont_ref[...],
                            preferred_element_type=jnp.float32) + b_cont_ref[...])

    cat = x_cat_ref[...]
    chrg = cat[:, 1:2] + 1
    pdg = jnp.abs(cat[:, 0:1])
    for i, p in enumerate(_PDGS):
        pdg = jnp.where(pdg == p, jnp.full_like(pdg, i), pdg)

    emb_chrg = jnp.zeros((_N, _H // 4), jnp.float32)
    for k in range(3):
        emb_chrg += jnp.where(chrg == k, 1.0, 0.0) * emb_charge_ref[k, :][None, :]
    emb_pdg = jnp.zeros((_N, _H // 4), jnp.float32)
    for k in range(7):
        emb_pdg += jnp.where(pdg == k, 1.0, 0.0) * emb_pdgid_ref[k, :][None, :]

    w_cat = w_cat_ref[...]
    emb_cat = _elu(jnp.dot(emb_chrg, w_cat[:_H // 4, :],
                           preferred_element_type=jnp.float32)
                   + jnp.dot(emb_pdg, w_cat[_H // 4:, :],
                             preferred_element_type=jnp.float32)
                   + b_cat_ref[...])

    w_enc = w_enc_ref[...]
    enc = _elu(jnp.dot(emb_cat, w_enc[:_H // 2, :],
                       preferred_element_type=jnp.float32)
               + jnp.dot(emb_cont, w_enc[_H // 2:, :],
                         preferred_element_type=jnp.float32)
               + b_enc_ref[...])
    emb = _bn(enc, g_all_ref[...], b_all_ref[...])
    emb_ref[...] = emb

    w_msg = w_msg_ref[...]
    wt = w_msg[:_H, :]
    wb = w_msg[_H:, :]
    a_ref[...] = jnp.dot(emb, wt - wb, preferred_element_type=jnp.float32) + b_msg_ref[...]
    bt_ref[...] = jnp.dot(emb, wb, preferred_element_type=jnp.float32).T


_enc_call = pl.pallas_call(
    _enc_body,
    out_shape=[
        jax.ShapeDtypeStruct((_N, _H), jnp.float32),
        jax.ShapeDtypeStruct((_N, _H), jnp.float32),
        jax.ShapeDtypeStruct((_H, _N), jnp.float32),
    ],
)


def _segmax_body(bt_hbm, e_hbm, out_hbm, b_buf, m_buf, e0, e1, sem0, sem1):
    cid = lax.axis_index("c")
    sid = lax.axis_index("s")
    wid = sid * 2 + cid
    f0 = wid * _FPT

    pltpu.sync_copy(bt_hbm.at[pl.ds(f0, _FPT), :], b_buf)

    neg = jnp.full((_LANES,), -jnp.inf, jnp.float32)
    def _init(i, c):
        for f in range(_FPT):
            m_buf[f, pl.ds(i * _LANES, _LANES)] = neg
        return c
    lax.fori_loop(0, _N // _LANES, _init, 0)

    fvecs = [jnp.full((_LANES,), f, jnp.int32) for f in range(_FPT)]

    def _start(ci, buf, sem):
        pltpu.async_copy(e_hbm.at[:, pl.ds(ci * _CH, _CH)], buf, sem)

    def _wait(ci, buf, sem):
        pltpu.make_async_copy(e_hbm.at[:, pl.ds(ci * _CH, _CH)], buf,
                              sem).wait()

    def _process(e_buf):
        def _vec(vi, cc):
            base_v = vi * (_G * _LANES)
            ds = [e_buf[1, pl.ds(base_v + g * _LANES, _LANES)]
                  for g in range(_G)]
            ss = [e_buf[0, pl.ds(base_v + g * _LANES, _LANES)]
                  for g in range(_G)]
            cms = [plsc.scan_count(d) for d in ds]
            bss = [[plsc.load_gather(b_buf, [fv, s]) for fv in fvecs]
                   for s in ss]
            clean = cms[0][1]
            for _, l in cms[1:]:
                clean = clean & l
            for (cnt, _), d, bs in zip(cms, ds, bss):
                msk1 = cnt == 1
                for fv, b in zip(fvecs, bs):
                    cur = plsc.load_gather(m_buf, [fv, d])
                    plsc.store_scatter(m_buf, [fv, d], jnp.maximum(cur, b),
                                       mask=msk1)

            def _slow(_):
                for (cnt, _), d, bs in zip(cms, ds, bss):
                    nmax = jnp.max(cnt)

                    def _round(r, c2):
                        msk = cnt == r
                        for fv, b in zip(fvecs, bs):
                            cur = plsc.load_gather(m_buf, [fv, d])
                            plsc.store_scatter(m_buf, [fv, d],
                                               jnp.maximum(cur, b), mask=msk)
                        return c2
                    lax.fori_loop(2, nmax + 1, _round, 0)
                return 0
            lax.cond(jnp.all(clean), lambda _: 0, _slow, 0)
            return cc
        lax.fori_loop(0, _CH // (_G * _LANES), _vec, 0)

    npairs = _E // _CH // 2
    _start(0, e0, sem0)
    _start(1, e1, sem1)

    def _pair(p, c):
        c0 = 2 * p
        _wait(c0, e0, sem0)
        _process(e0)

        @pl.when(p < npairs - 1)
        def _():
            _start(c0 + 2, e0, sem0)
        _wait(c0 + 1, e1, sem1)
        _process(e1)

        @pl.when(p < npairs - 1)
        def _():
            _start(c0 + 3, e1, sem1)
        return c

    lax.fori_loop(0, npairs, _pair, 0)

    pltpu.sync_copy(m_buf, out_hbm.at[pl.ds(f0, _FPT), :])


_segmax_call = pl.kernel(
    _segmax_body,
    out_type=jax.ShapeDtypeStruct((_H, _N), jnp.float32),
    mesh=plsc.VectorSubcoreMesh(core_axis_name="c", subcore_axis_name="s"),
    compiler_params=pltpu.CompilerParams(needs_layout_passes=False),
    scratch_types=[
        pltpu.VMEM((_FPT, _N), jnp.float32),
        pltpu.VMEM((_FPT, _N), jnp.float32),
        pltpu.VMEM((2, _CH), jnp.int32),
        pltpu.VMEM((2, _CH), jnp.int32),
        pltpu.SemaphoreType.DMA,
        pltpu.SemaphoreType.DMA,
    ],
)


def _tail_body(emb_ref, a_ref, mt_ref, g1_ref, b1_ref,
               w_o1_ref, b_o1_ref, w_o2_ref, b_o2_ref, out_ref):
    m = mt_ref[...].T
    agg = jnp.where(m > -jnp.inf, a_ref[...] + m, 0.0)
    emb2 = emb_ref[...] + _bn(agg, g1_ref[...], b1_ref[...])
    h = _elu(jnp.dot(emb2, w_o1_ref[...], preferred_element_type=jnp.float32)
             + b_o1_ref[...])
    out_ref[...] = jnp.dot(h, w_o2_ref[...],
                           preferred_element_type=jnp.float32) + b_o2_ref[...]


_tail_call = pl.pallas_call(
    _tail_body,
    out_shape=jax.ShapeDtypeStruct((_N, 1), jnp.float32),
)


def kernel(x_cont, x_cat, edge_index, batch, emb_charge, emb_pdgid,
           W_cont, b_cont, W_cat, b_cat, W_enc, b_enc, g_all, b_all,
           W_msg, b_msg, g_bn1, b_bn1, W_o1, b_o1, W_o2, b_o2):
    del batch
    emb, a, bt = _enc_call(x_cont, x_cat, emb_charge, emb_pdgid,
                           W_cont, b_cont, W_cat, b_cat, W_enc, b_enc,
                           g_all, b_all, W_msg, b_msg)
    mt = _segmax_call(bt, edge_index)
    out = _tail_call(emb, a, mt, g_bn1, b_bn1, W_o1, b_o1, W_o2, b_o2)
    return out.squeeze(-1)
